# Initial kernel scaffold; baseline (speedup 1.0000x reference)
import jax
import jax.numpy as jnp
from jax.experimental import pallas as pl


def kernel(x, knn_edge_index, genet_edge_index, Wl_c, Wr_c, b_c, Wl_r, Wr_r, b_r):
    raise NotImplementedError("write your pallas kernel here")


# speedup vs baseline: 38.6416x; 38.6416x over previous
import functools

import jax
import jax.numpy as jnp
from jax import lax
from jax.experimental import pallas as pl
from jax.experimental.pallas import tpu as pltpu
from jax.experimental.pallas import tpu_sc as plsc

_COL = 1024
_ROW = 512
_LAYERS = 3
_EK = 16384
_EG = 65536

_NC = 2
_NS = 16
_NW = _NC * _NS

_ZB = 16384


def _sc_count_body(kd, ks, gd, gs, out_k, out_g, sh_k, sh_g,
                   zbuf, ones_v, idx_v, dvm, svm):
    c = lax.axis_index("c")
    s = lax.axis_index("s")
    zero16 = jnp.zeros((16,), jnp.float32)
    one16 = jnp.ones((16,), jnp.float32)

    def zfill(i, carry):
        zbuf[pl.ds(i * 16, 16)] = zero16
        return carry

    lax.fori_loop(0, _ZB // 16, zfill, 0)
    for j in range(8):
        ones_v[pl.ds(j * 16, 16)] = one16

    kslice = (_ROW * _ROW) // _NS
    gslice = (_COL * _COL) // _NS
    pltpu.sync_copy(zbuf.at[pl.ds(0, kslice)],
                    sh_k.at[pl.ds(s * kslice, kslice)])
    for r in range(gslice // _ZB):
        pltpu.sync_copy(zbuf, sh_g.at[pl.ds(s * gslice + r * _ZB, _ZB)])
    plsc.subcore_barrier()

    def scatter_graph(dref, sref, shared, n, e_total, swap):
        e_tile = e_total // _NW
        base = (c * _NS + s) * e_tile
        pltpu.sync_copy(dref.at[pl.ds(base, e_tile)], dvm.at[pl.ds(0, e_tile)])
        pltpu.sync_copy(sref.at[pl.ds(base, e_tile)], svm.at[pl.ds(0, e_tile)])
        for chunk in range(e_tile // 128):
            for j in range(8):
                off = chunk * 128 + j * 16
                d = dvm[pl.ds(off, 16)]
                sv = svm[pl.ds(off, 16)]
                flat = sv * n + d if swap else d * n + sv
                idx_v[pl.ds(j * 16, 16)] = flat
            pltpu.sync_copy(ones_v, shared.at[idx_v], add=True)

    scatter_graph(kd, ks, sh_k, _ROW, _EK, swap=True)
    scatter_graph(gd, gs, sh_g, _COL, _EG, swap=False)
    plsc.subcore_barrier()

    pltpu.sync_copy(sh_k.at[pl.ds(s * kslice, kslice)],
                    out_k.at[c, pl.ds(s * kslice, kslice)])
    for r in range(gslice // _ZB):
        off = s * gslice + r * _ZB
        pltpu.sync_copy(sh_g.at[pl.ds(off, _ZB)], out_g.at[c, pl.ds(off, _ZB)])


def _sc_counts(kd, ks, gd, gs):
    mesh = plsc.VectorSubcoreMesh(core_axis_name="c", subcore_axis_name="s")
    f32 = jnp.float32
    run = functools.partial(
        pl.kernel,
        mesh=mesh,
        out_type=[
            jax.ShapeDtypeStruct((_NC, _ROW * _ROW), f32),
            jax.ShapeDtypeStruct((_NC, _COL * _COL), f32),
        ],
        scratch_types=[
            pltpu.VMEM_SHARED((_ROW * _ROW,), f32),
            pltpu.VMEM_SHARED((_COL * _COL,), f32),
            pltpu.VMEM((_ZB,), f32),
            pltpu.VMEM((128,), f32),
            pltpu.VMEM((128,), jnp.int32),
            pltpu.VMEM((_EG // _NW,), jnp.int32),
            pltpu.VMEM((_EG // _NW,), jnp.int32),
        ],
    )(_sc_count_body)
    return run(kd, ks, gd, gs)


def _tc_net_body(ctk_ref, cg_ref, x_ref, wlc_ref, wrc_ref, bc_ref,
                 wlrT_ref, wrrT_ref, br_ref, out_ref, e_s, bk_s, ag_s):
    i = pl.program_id(0)
    f32 = jnp.float32

    @pl.when(i == 0)
    def _init():
        ctk = ctk_ref[0] + ctk_ref[1]
        cnt_k = jnp.sum(ctk, axis=0, keepdims=True)
        bk_s[...] = ctk / jnp.maximum(cnt_k, 1.0)
        cg = cg_ref[0] + cg_ref[1]
        cnt_g = jnp.sum(cg, axis=1, keepdims=True)
        ag_s[...] = cg / jnp.maximum(cnt_g, 1.0)
        e_s[...] = x_ref[...]

    e = e_s[...]
    t = jnp.dot(e, bk_s[...], preferred_element_type=f32)
    h = jnp.dot(wlc_ref[0], t, preferred_element_type=f32)
    h += jnp.dot(wrc_ref[0], e, preferred_element_type=f32)
    h += bc_ref[0]
    e1 = jnp.where(h >= 0, h, h * 0.01)
    u = jnp.dot(ag_s[...], e1, preferred_element_type=f32)
    h2 = jnp.dot(u, wlrT_ref[0], preferred_element_type=f32)
    h2 += jnp.dot(e1, wrrT_ref[0], preferred_element_type=f32)
    h2 += br_ref[0]
    e2 = jnp.where(h2 >= 0, h2, h2 * 0.01)
    e_s[...] = e2

    @pl.when(i == _LAYERS - 1)
    def _fin():
        out_ref[...] = e2


def _tc_forward(ctk, cg, x, Wl_c, Wr_c, bc, wlrT, wrrT, br):
    f32 = jnp.float32
    return pl.pallas_call(
        _tc_net_body,
        grid=(_LAYERS,),
        in_specs=[
            pl.BlockSpec((_NC, _ROW, _ROW), lambda i: (0, 0, 0)),
            pl.BlockSpec((_NC, _COL, _COL), lambda i: (0, 0, 0)),
            pl.BlockSpec((_COL, _ROW), lambda i: (0, 0)),
            pl.BlockSpec((1, _COL, _COL), lambda i: (i, 0, 0)),
            pl.BlockSpec((1, _COL, _COL), lambda i: (i, 0, 0)),
            pl.BlockSpec((1, _COL, 1), lambda i: (i, 0, 0)),
            pl.BlockSpec((1, _ROW, _ROW), lambda i: (i, 0, 0)),
            pl.BlockSpec((1, _ROW, _ROW), lambda i: (i, 0, 0)),
            pl.BlockSpec((1, 1, _ROW), lambda i: (i, 0, 0)),
        ],
        out_specs=pl.BlockSpec((_COL, _ROW), lambda i: (0, 0)),
        out_shape=jax.ShapeDtypeStruct((_COL, _ROW), f32),
        scratch_shapes=[
            pltpu.VMEM((_COL, _ROW), f32),
            pltpu.VMEM((_ROW, _ROW), f32),
            pltpu.VMEM((_COL, _COL), f32),
        ],
        compiler_params=pltpu.CompilerParams(
            dimension_semantics=("arbitrary",),
        ),
    )(ctk, cg, x, Wl_c, Wr_c, bc, wlrT, wrrT, br)


def kernel(x, knn_edge_index, genet_edge_index, Wl_c, Wr_c, b_c,
           Wl_r, Wr_r, b_r):
    kd = knn_edge_index[1]
    ks = knn_edge_index[0]
    gd = genet_edge_index[1]
    gs = genet_edge_index[0]
    ck2, cg2 = _sc_counts(kd, ks, gd, gs)
    ctk = ck2.reshape(_NC, _ROW, _ROW)
    cg = cg2.reshape(_NC, _COL, _COL)
    wlrT = Wl_r.transpose(0, 2, 1)
    wrrT = Wr_r.transpose(0, 2, 1)
    bc = b_c[:, :, None]
    br = b_r[:, None, :]
    return _tc_forward(ctk, cg, x, Wl_c, Wr_c, bc, wlrT, wrrT, br)


# speedup vs baseline: 44.4099x; 1.1493x over previous
import functools

import jax
import jax.numpy as jnp
from jax import lax
from jax.experimental import pallas as pl
from jax.experimental.pallas import tpu as pltpu
from jax.experimental.pallas import tpu_sc as plsc

_COL = 1024
_ROW = 512
_LAYERS = 3
_EK = 16384
_EG = 65536

_NC = 2
_NS = 16
_NW = _NC * _NS

_ZB = 4096


def _sc_count_body(kd, ks, gd, gs, out_k, out_g, sh_k, sh_g,
                   zbuf, ones_v, idx_a, idx_b, dvm_k, svm_k, dvm_g, svm_g,
                   sem_e, sem_z, sem_a, sem_b):
    c = lax.axis_index("c")
    s = lax.axis_index("s")
    zero16 = jnp.zeros((16,), jnp.float32)
    one16 = jnp.ones((16,), jnp.float32)

    ek = _EK // _NW
    eg = _EG // _NW
    base_k = (c * _NS + s) * ek
    base_g = (c * _NS + s) * eg
    loads = [
        pltpu.async_copy(kd.at[pl.ds(base_k, ek)], dvm_k, sem_e),
        pltpu.async_copy(ks.at[pl.ds(base_k, ek)], svm_k, sem_e),
        pltpu.async_copy(gd.at[pl.ds(base_g, eg)], dvm_g, sem_e),
        pltpu.async_copy(gs.at[pl.ds(base_g, eg)], svm_g, sem_e),
    ]

    def zfill(i, carry):
        zbuf[pl.ds(i * 16, 16)] = zero16
        return carry

    lax.fori_loop(0, _ZB // 16, zfill, 0)
    for j in range(8):
        ones_v[pl.ds(j * 16, 16)] = one16

    kslice = (_ROW * _ROW) // _NS
    gslice = (_COL * _COL) // _NS
    zeros = []
    for r in range(kslice // _ZB):
        zeros.append(pltpu.async_copy(
            zbuf, sh_k.at[pl.ds(s * kslice + r * _ZB, _ZB)], sem_z))
    for r in range(gslice // _ZB):
        zeros.append(pltpu.async_copy(
            zbuf, sh_g.at[pl.ds(s * gslice + r * _ZB, _ZB)], sem_z))
    for h in loads:
        h.wait()
    for h in zeros:
        h.wait()
    plsc.subcore_barrier()

    def scatter_graph(dvm, svm, shared, n, e_tile):
        bufs = (idx_a, idx_b)
        sems = (sem_a, sem_b)
        handles = [None, None]
        for chunk in range(e_tile // 128):
            p = chunk % 2
            if handles[p] is not None:
                handles[p].wait()
            for j in range(8):
                off = chunk * 128 + j * 16
                d = dvm[pl.ds(off, 16)]
                sv = svm[pl.ds(off, 16)]
                bufs[p][pl.ds(j * 16, 16)] = d * n + sv
            handles[p] = pltpu.async_copy(
                ones_v, shared.at[bufs[p]], sems[p], add=True)
        for h in handles:
            if h is not None:
                h.wait()

    scatter_graph(svm_k, dvm_k, sh_k, _ROW, ek)
    scatter_graph(dvm_g, svm_g, sh_g, _COL, eg)
    plsc.subcore_barrier()

    pltpu.sync_copy(sh_k.at[pl.ds(s * kslice, kslice)],
                    out_k.at[pl.ds(c * _ROW * _ROW + s * kslice, kslice)])
    pltpu.sync_copy(sh_g.at[pl.ds(s * gslice, gslice)],
                    out_g.at[pl.ds(c * _COL * _COL + s * gslice, gslice)])


def _sc_counts(kd, ks, gd, gs):
    mesh = plsc.VectorSubcoreMesh(core_axis_name="c", subcore_axis_name="s")
    f32 = jnp.float32
    i32 = jnp.int32
    run = functools.partial(
        pl.kernel,
        mesh=mesh,
        out_type=[
            jax.ShapeDtypeStruct((_NC * _ROW * _ROW,), f32),
            jax.ShapeDtypeStruct((_NC * _COL * _COL,), f32),
        ],
        scratch_types=[
            pltpu.VMEM_SHARED((_ROW * _ROW,), f32),
            pltpu.VMEM_SHARED((_COL * _COL,), f32),
            pltpu.VMEM((_ZB,), f32),
            pltpu.VMEM((128,), f32),
            pltpu.VMEM((128,), i32),
            pltpu.VMEM((128,), i32),
            pltpu.VMEM((_EK // _NW,), i32),
            pltpu.VMEM((_EK // _NW,), i32),
            pltpu.VMEM((_EG // _NW,), i32),
            pltpu.VMEM((_EG // _NW,), i32),
            pltpu.SemaphoreType.DMA,
            pltpu.SemaphoreType.DMA,
            pltpu.SemaphoreType.DMA,
            pltpu.SemaphoreType.DMA,
        ],
    )(_sc_count_body)
    return run(kd, ks, gd, gs)


def _tc_net_body(ctk_ref, cg_ref, x_ref, wlc_ref, wrc_ref, bc_ref,
                 wlrT_ref, wrrT_ref, br_ref, out_ref, e_s, bk_s, ag_s):
    i = pl.program_id(0)
    f32 = jnp.float32

    @pl.when(i == 0)
    def _init():
        ctk = ctk_ref[0] + ctk_ref[1]
        cnt_k = jnp.sum(ctk, axis=0, keepdims=True)
        bk_s[...] = ctk / jnp.maximum(cnt_k, 1.0)
        cg = cg_ref[0] + cg_ref[1]
        cnt_g = jnp.sum(cg, axis=1, keepdims=True)
        ag_s[...] = cg / jnp.maximum(cnt_g, 1.0)
        e_s[...] = x_ref[...]

    e = e_s[...]
    t = jnp.dot(e, bk_s[...], preferred_element_type=f32)
    h = jnp.dot(wlc_ref[0], t, preferred_element_type=f32)
    h += jnp.dot(wrc_ref[0], e, preferred_element_type=f32)
    h += bc_ref[0]
    e1 = jnp.where(h >= 0, h, h * 0.01)
    u = jnp.dot(ag_s[...], e1, preferred_element_type=f32)
    h2 = jnp.dot(u, wlrT_ref[0], preferred_element_type=f32)
    h2 += jnp.dot(e1, wrrT_ref[0], preferred_element_type=f32)
    h2 += br_ref[0]
    e2 = jnp.where(h2 >= 0, h2, h2 * 0.01)
    e_s[...] = e2

    @pl.when(i == _LAYERS - 1)
    def _fin():
        out_ref[...] = e2


def _tc_forward(ctk, cg, x, Wl_c, Wr_c, bc, wlrT, wrrT, br):
    f32 = jnp.float32
    return pl.pallas_call(
        _tc_net_body,
        grid=(_LAYERS,),
        in_specs=[
            pl.BlockSpec((_NC, _ROW, _ROW), lambda i: (0, 0, 0)),
            pl.BlockSpec((_NC, _COL, _COL), lambda i: (0, 0, 0)),
            pl.BlockSpec((_COL, _ROW), lambda i: (0, 0)),
            pl.BlockSpec((1, _COL, _COL), lambda i: (i, 0, 0)),
            pl.BlockSpec((1, _COL, _COL), lambda i: (i, 0, 0)),
            pl.BlockSpec((1, _COL, 1), lambda i: (i, 0, 0)),
            pl.BlockSpec((1, _ROW, _ROW), lambda i: (i, 0, 0)),
            pl.BlockSpec((1, _ROW, _ROW), lambda i: (i, 0, 0)),
            pl.BlockSpec((1, 1, _ROW), lambda i: (i, 0, 0)),
        ],
        out_specs=pl.BlockSpec((_COL, _ROW), lambda i: (0, 0)),
        out_shape=jax.ShapeDtypeStruct((_COL, _ROW), f32),
        scratch_shapes=[
            pltpu.VMEM((_COL, _ROW), f32),
            pltpu.VMEM((_ROW, _ROW), f32),
            pltpu.VMEM((_COL, _COL), f32),
        ],
        compiler_params=pltpu.CompilerParams(
            dimension_semantics=("arbitrary",),
        ),
    )(ctk, cg, x, Wl_c, Wr_c, bc, wlrT, wrrT, br)


def kernel(x, knn_edge_index, genet_edge_index, Wl_c, Wr_c, b_c,
           Wl_r, Wr_r, b_r):
    kd = knn_edge_index[1]
    ks = knn_edge_index[0]
    gd = genet_edge_index[1]
    gs = genet_edge_index[0]
    ck2, cg2 = _sc_counts(kd, ks, gd, gs)
    ctk = ck2.reshape(_NC, _ROW, _ROW)
    cg = cg2.reshape(_NC, _COL, _COL)
    wlrT = Wl_r.transpose(0, 2, 1)
    wrrT = Wr_r.transpose(0, 2, 1)
    bc = b_c[:, :, None]
    br = b_r[:, None, :]
    return _tc_forward(ctk, cg, x, Wl_c, Wr_c, bc, wlrT, wrrT, br)
